# baseline (speedup 1.0000x reference)
import functools

import jax
import jax.numpy as jnp
from jax import lax
from jax.experimental import pallas as pl
from jax.experimental.pallas import tpu as pltpu
from jax.experimental.pallas import tpu_sc as plsc

VOCAB = 1000000
DIM = 64
B = 16384
NEG = 5

NC = 2
NS = 16
NW = NC * NS

CHUNK = 512
POS_PER_W = B // NW
NEG_PER_W = (B * NEG) // NW
NEG_CHUNKS = NEG_PER_W // CHUNK
N_CHUNKS = 2 + NEG_CHUNKS


def _body(iw, ow, ng, tin, tout, o_in, o_out, o_neg, *rest):
    idx_bufs = rest[:N_CHUNKS]
    rows_v, sem = rest[N_CHUNKS], rest[N_CHUNKS + 1]
    wid = lax.axis_index("s") * NC + lax.axis_index("c")
    pos_base = wid * POS_PER_W
    neg_base = wid * NEG_PER_W

    pltpu.sync_copy(iw.at[pl.ds(pos_base, CHUNK)], idx_bufs[0])
    pltpu.sync_copy(ow.at[pl.ds(pos_base, CHUNK)], idx_bufs[1])
    for c in range(NEG_CHUNKS):
        pltpu.sync_copy(ng.at[pl.ds(neg_base + c * CHUNK, CHUNK)],
                        idx_bufs[2 + c])

    tasks = [(tin, 0, o_in, pos_base), (tout, 1, o_out, pos_base)]
    tasks += [(tout, 2 + c, o_neg, neg_base + c * CHUNK)
              for c in range(NEG_CHUNKS)]

    for table, row, out, base in tasks:
        pltpu.async_copy(table.at[idx_bufs[row]], rows_v, sem).wait()
        pltpu.sync_copy(rows_v, out.at[pl.ds(base, CHUNK)])


_sc_gather = functools.partial(
    pl.kernel,
    out_type=[
        jax.ShapeDtypeStruct((B, DIM), jnp.float32),
        jax.ShapeDtypeStruct((B, DIM), jnp.float32),
        jax.ShapeDtypeStruct((B * NEG, DIM), jnp.float32),
    ],
    mesh=plsc.VectorSubcoreMesh(
        core_axis_name="c", subcore_axis_name="s",
        num_cores=NC, num_subcores=NS),
    compiler_params=pltpu.CompilerParams(use_tc_tiling_on_sc=False),
    scratch_types=(
        [pltpu.VMEM((CHUNK,), jnp.int32) for _ in range(N_CHUNKS)]
        + [pltpu.VMEM((CHUNK, DIM), jnp.float32),
           pltpu.SemaphoreType.DMA]
    ),
)(_body)


def kernel(input_words, output_words, neg_words, in_table, out_table):
    o_in, o_out, o_neg = _sc_gather(
        input_words.astype(jnp.int32), output_words.astype(jnp.int32),
        neg_words.astype(jnp.int32), in_table, out_table)
    return o_in, o_out, o_neg.reshape(B, NEG, DIM)
